# baseline (speedup 1.0000x reference)
import functools

import jax
import jax.numpy as jnp
import numpy as np
from jax import lax
from jax.experimental import pallas as pl
from jax.experimental.pallas import tpu as pltpu
from jax.experimental.pallas import tpu_sc as plsc

B = 64
F = 599
T = 512
FC_IDX = 219
MID_LO = 199
MID_HI = 239
NOISE_COUNT = (F - (MID_HI - MID_LO + 1)) * T

RF1 = 16
G1 = (F + RF1 - 1) // RF1
RF3 = 16
G3 = (F + RF3 - 1) // RF3


def _stats_kernel(x_ref, rowfc_ref, sums_ref):
    i = pl.program_id(0)
    f = lax.broadcasted_iota(jnp.int32, (RF1, 8, 128), 0) + i * RF1
    a = jnp.abs(x_ref[...])
    zero = jnp.zeros_like(a)
    valid = (f < F)[None]
    midm = ((f >= MID_LO) & (f <= MID_HI))[None]
    fcm = (f == FC_IDX)[None]
    s_all = jnp.sum(jnp.where(valid, a, zero), axis=(1, 2, 3))
    s_mid = jnp.sum(jnp.where(midm, a, zero), axis=(1, 2, 3))
    fc_part = jnp.sum(jnp.where(fcm, a, zero), axis=1)

    @pl.when(i == 0)
    def _():
        rowfc_ref[...] = fc_part
        sums_ref[:, 0:1] = s_all[:, None]
        sums_ref[:, 1:2] = s_mid[:, None]
        sums_ref[:, 2:16] = jnp.zeros((B, 14), jnp.float32)

    @pl.when(i > 0)
    def _():
        rowfc_ref[...] = rowfc_ref[...] + fc_part
        sums_ref[:, 0:1] = sums_ref[:, 0:1] + s_all[:, None]
        sums_ref[:, 1:2] = sums_ref[:, 1:2] + s_mid[:, None]


_HB_KMAX = 380
_THR = np.full((384,), np.inf, np.float32)
_THR[: _HB_KMAX - 8] = (
    10.0 ** ((np.arange(9, _HB_KMAX + 1) + 261.0) / 60.0)
).astype(np.float32)

_NW = 16
_IPW = B // _NW


def _sc_mask_kernel(rowfc_hbm, sums_hbm, thr_hbm, h_hbm, band_hbm, stage_hbm,
                    slab_v, sums_v, mag_v, thr_v, pub_v, all_v, h4_v, hv_v,
                    band_v, tmpf_v, tmpi_v):
    core = lax.axis_index("c")
    wid = lax.axis_index("s")
    ivec = lax.iota(jnp.int32, 16)

    def _allred(tmp_ref, v, op):
        del tmp_ref
        s = v[0]
        for i in range(1, 16):
            s = op(s, v[i])
        return jnp.broadcast_to(s, (16,))

    @pl.when(core == 0)
    def _():
        pltpu.sync_copy(thr_hbm, thr_v)
        for k in range(_IPW):
            b = wid * _IPW + k
            pltpu.sync_copy(rowfc_hbm.at[b], slab_v)
            pltpu.sync_copy(sums_hbm.at[b], sums_v)
            mvec = jnp.full((16,), -jnp.inf, jnp.float32)
            for g in range(4):
                for j in range(8):
                    mag = (slab_v[2 * g, pl.ds(16 * j, 16)]
                           + slab_v[2 * g + 1, pl.ds(16 * j, 16)])
                    mag_v[pl.ds(g * 128 + 16 * j, 16)] = mag
                    mvec = jnp.maximum(mvec, mag)
            m = _allred(tmpf_v, mvec, jnp.maximum)[0]
            minv = jnp.full((16,), T, jnp.int32)
            for j in range(32):
                c = mag_v[pl.ds(16 * j, 16)]
                tv = ivec + 16 * j
                minv = jnp.minimum(minv, jnp.where(c == m, tv, T))
            mid = _allred(tmpi_v, minv, jnp.minimum)[0]
            lo20 = jnp.maximum(mid - 20, 0)
            hi20 = mid + 20
            sigv = jnp.zeros((16,), jnp.float32)
            for j in range(32):
                c = mag_v[pl.ds(16 * j, 16)]
                tv = ivec + 16 * j
                sigv = sigv + jnp.where((tv >= lo20) & (tv < hi20), c,
                                        jnp.zeros((16,), jnp.float32))
            sig = _allred(tmpf_v, sigv, jnp.add)[0]
            sv = sums_v[...]
            s_all = sv[0]
            s_mid = sv[1]
            noise = (s_all - s_mid) * (1.0 / NOISE_COUNT)
            d = jnp.broadcast_to(sig - noise, (16,))
            nv = jnp.broadcast_to(noise, (16,))
            r = (d * d) / (nv * nv)
            cntv = jnp.zeros((16,), jnp.int32)
            one16 = jnp.ones((16,), jnp.int32)
            zero16 = jnp.zeros((16,), jnp.int32)
            for j in range(24):
                thr = thr_v[pl.ds(16 * j, 16)]
                cntv = cntv + jnp.where(r >= thr, one16, zero16)
            hb = 8 + _allred(tmpi_v, cntv, jnp.add)[0]
            pub_v[...] = jnp.where(ivec == 0, mid,
                                   jnp.where(ivec == 1, hb,
                                             jnp.zeros((16,), jnp.int32)))
            pltpu.sync_copy(pub_v, stage_hbm.at[b])
        plsc.subcore_barrier()

        @pl.when(wid == 0)
        def _():
            pltpu.sync_copy(stage_hbm, all_v)
            for j in range(32):
                h4_v[pl.ds(16 * j, 16)] = jnp.zeros((16,), jnp.int32)

            def body(b, hbmax):
                row = all_v[b, pl.ds(0, 16)]
                mid = row[0]
                hb = row[1]
                lo = jnp.maximum(mid - 8, 0)
                hi = jnp.minimum(mid + 8, T)
                c0 = lax.shift_right_logical(lo, 4)
                for dcc in range(2):
                    cc = jnp.minimum(c0 + dcc, 31)
                    chunk = h4_v[pl.ds(cc * 16, 16)]
                    tv = ivec + cc * 16
                    upd = jnp.where((tv >= lo) & (tv < hi),
                                    jnp.maximum(chunk, hb), chunk)
                    h4_v[pl.ds(cc * 16, 16)] = upd
                return jnp.maximum(hbmax, hb)

            hbmax = lax.fori_loop(0, B, body, jnp.int32(8))
            for g in range(4):
                for j in range(8):
                    chunk = h4_v[pl.ds(g * 128 + 16 * j, 16)]
                    hv_v[2 * g, pl.ds(16 * j, 16)] = chunk
                    hv_v[2 * g + 1, pl.ds(16 * j, 16)] = chunk
            lo_blk = lax.shift_right_logical(jnp.maximum(FC_IDX - hbmax, 0), 4)
            hi_blk = lax.shift_right_logical(
                jnp.minimum(FC_IDX + hbmax, F) - 1, 4)
            band_v[...] = jnp.where(ivec == 0, lo_blk,
                                    jnp.where(ivec == 1, hi_blk,
                                              jnp.zeros((16,), jnp.int32)))
            pltpu.sync_copy(hv_v, h_hbm)
            pltpu.sync_copy(band_v, band_hbm)


def _apply_kernel(band_ref, x_ref, h_ref, out_ref):
    i = pl.program_id(0)
    f = lax.broadcasted_iota(jnp.int32, (RF3, 8, 128), 0) + i * RF3
    h = h_ref[...][None]
    mask = (f >= FC_IDX - h) & (f < FC_IDX + h)
    x = x_ref[...]
    out_ref[...] = jnp.where(mask[None], x, jnp.zeros_like(x))


def kernel(x):
    shape = x.shape
    xv = (x.reshape(16, 4, 2, F, 4, 128)
          .transpose(0, 1, 3, 4, 2, 5)
          .reshape(B, F, 8, 128))

    row_fc, sums = pl.pallas_call(
        _stats_kernel,
        grid=(G1,),
        in_specs=[pl.BlockSpec((B, RF1, 8, 128), lambda i: (0, i, 0, 0))],
        out_specs=[
            pl.BlockSpec((B, 8, 128), lambda i: (0, 0, 0)),
            pl.BlockSpec((B, 16), lambda i: (0, 0)),
        ],
        out_shape=[
            jax.ShapeDtypeStruct((B, 8, 128), jnp.float32),
            jax.ShapeDtypeStruct((B, 16), jnp.float32),
        ],
    )(xv)

    sc_mask = functools.partial(
        pl.kernel,
        mesh=plsc.VectorSubcoreMesh(core_axis_name="c", subcore_axis_name="s"),
        out_type=[
            jax.ShapeDtypeStruct((8, 128), jnp.int32),
            jax.ShapeDtypeStruct((16,), jnp.int32),
            jax.ShapeDtypeStruct((B, 16), jnp.int32),
        ],
        scratch_types=[
            pltpu.VMEM((8, 128), jnp.float32),
            pltpu.VMEM((16,), jnp.float32),
            pltpu.VMEM((512,), jnp.float32),
            pltpu.VMEM((384,), jnp.float32),
            pltpu.VMEM((16,), jnp.int32),
            pltpu.VMEM((B, 16), jnp.int32),
            pltpu.VMEM((512,), jnp.int32),
            pltpu.VMEM((8, 128), jnp.int32),
            pltpu.VMEM((16,), jnp.int32),
            pltpu.VMEM((32,), jnp.float32),
            pltpu.VMEM((32,), jnp.int32),
        ],
    )(_sc_mask_kernel)
    h, band, _ = sc_mask(row_fc, sums, jnp.asarray(_THR))

    out = pl.pallas_call(
        _apply_kernel,
        grid_spec=pltpu.PrefetchScalarGridSpec(
            num_scalar_prefetch=1,
            grid=(G3,),
            in_specs=[
                pl.BlockSpec((B, RF3, 8, 128),
                             lambda i, b: (0, jnp.clip(i, b[0], b[1]), 0, 0)),
                pl.BlockSpec((8, 128), lambda i, b: (0, 0)),
            ],
            out_specs=pl.BlockSpec((B, RF3, 8, 128),
                                   lambda i, b: (0, i, 0, 0)),
        ),
        out_shape=jax.ShapeDtypeStruct((B, F, 8, 128), jnp.float32),
    )(band, xv, h)

    return (out.reshape(16, 4, F, 4, 2, 128)
            .transpose(0, 1, 4, 2, 3, 5)
            .reshape(shape))
